# Initial kernel scaffold; baseline (speedup 1.0000x reference)
import jax
import jax.numpy as jnp
from jax.experimental import pallas as pl


def kernel(C, pos_table, story):
    raise NotImplementedError("write your pallas kernel here")


# speedup vs baseline: 1.2881x; 1.2881x over previous
import jax
import jax.numpy as jnp
from jax.experimental import pallas as pl

VOCAB = 100000
DIM = 128
HOPS = 3
POS = 200
PAD = 0
MEM = 200
BATCH = 1024
WIDTH = 4

_BB = 64


def _att_body(m1_ref, m2_ref, m3_ref, u_ref):
    m1 = m1_ref[...]
    m2 = m2_ref[...]
    m3 = m3_ref[...]
    u1 = jnp.mean(m1, axis=1)
    l1 = jnp.sum(m1 * u1[:, None, :], axis=2)
    p1 = jax.nn.softmax(l1, axis=1)
    u2 = u1 + jnp.sum(m2 * p1[:, :, None], axis=1)
    l2 = jnp.sum(m2 * u2[:, None, :], axis=2)
    p2 = jax.nn.softmax(l2, axis=1)
    u_ref[...] = u2 + jnp.sum(m3 * p2[:, :, None], axis=1)


def _attention(m1, m2, m3):
    B = m1.shape[0]
    grid = (B // _BB,)
    spec = pl.BlockSpec((_BB, MEM, DIM), lambda i: (i, 0, 0))
    return pl.pallas_call(
        _att_body,
        grid=grid,
        in_specs=[spec, spec, spec],
        out_specs=pl.BlockSpec((_BB, DIM), lambda i: (i, 0)),
        out_shape=jax.ShapeDtypeStruct((B, DIM), jnp.float32),
    )(m1, m2, m3)


def kernel(C, pos_table, story):
    s = jnp.transpose(story, (1, 0, 2))
    B = s.shape[0]
    Cz = C.at[:, PAD, :].set(0.0)
    pz = pos_table.at[PAD].set(0.0)
    padding_mask = s[:, :, 0] == PAD
    positions = jnp.cumsum((~padding_mask).astype(jnp.int32), axis=-1)
    positions = jnp.where(padding_mask, PAD, positions)
    ep = pz[positions]

    def pooled(k):
        g = jnp.take(Cz[k], s.reshape(B, -1), axis=0)
        return g.reshape(s.shape + (DIM,)).sum(2) + ep

    m1, m2, m3 = pooled(1), pooled(2), pooled(3)
    return _attention(m1, m2, m3)


# speedup vs baseline: 6.9933x; 5.4293x over previous
import jax
import jax.numpy as jnp
from jax import lax
from jax.experimental import pallas as pl
from jax.experimental.pallas import tpu as pltpu
from jax.experimental.pallas import tpu_sc as plsc

VOCAB = 100000
DIM = 128
POS = 200
PAD = 0
MEM = 200
BATCH = 1024
WIDTH = 4

NC, NS, L = 2, 16, 16
NW = NC * NS
BPW = BATCH // NW
NCHUNK = 8
CROWS = MEM * WIDTH // NCHUNK
MPC = MEM // NCHUNK
MPAD = 224

_PB = 128
_BB = 64


def _prep_body(sm_ref, pos_ref, n0_ref, pz_ref, tot_ref):
    sm = sm_ref[...]
    nonpad = sm[:, 0, :] != PAD
    npf = nonpad.astype(jnp.float32)
    ri = lax.broadcasted_iota(jnp.int32, (MPAD, MPAD), 0)
    ci = lax.broadcasted_iota(jnp.int32, (MPAD, MPAD), 1)
    tri = (ri <= ci).astype(jnp.float32)
    posf = jax.lax.dot_general(npf, tri, (((1,), (0,)), ((), ())),
                               preferred_element_type=jnp.float32)
    pos_ref[...] = jnp.where(nonpad, posf, 0.0).astype(jnp.int32)
    n0 = (sm == PAD).astype(jnp.float32).sum(axis=1)
    n0_ref[...] = n0
    pz_ref[...] = 1.0 - npf
    tot_ref[...] = jnp.broadcast_to(jnp.sum(n0, axis=1, keepdims=True),
                                    (sm.shape[0], L))


def _prep(sm):
    grid = (BATCH // _PB,)
    out_sh = [
        jax.ShapeDtypeStruct((BATCH, MPAD), jnp.int32),
        jax.ShapeDtypeStruct((BATCH, MPAD), jnp.float32),
        jax.ShapeDtypeStruct((BATCH, MPAD), jnp.float32),
        jax.ShapeDtypeStruct((BATCH, L), jnp.float32),
    ]
    ospec = pl.BlockSpec((_PB, MPAD), lambda i: (i, 0))
    return pl.pallas_call(
        _prep_body,
        grid=grid,
        in_specs=[pl.BlockSpec((_PB, WIDTH, MPAD), lambda i: (i, 0, 0))],
        out_specs=[ospec, ospec, ospec, pl.BlockSpec((_PB, L), lambda i: (i, 0))],
        out_shape=out_sh,
    )(sm)


def _sc_pool_body(cflat, pos_hbm, s3_hbm, posx_hbm, n0_hbm, pz_hbm, tot_hbm,
                  mout, sidx, posbuf, n0buf, pzbuf, totbuf, rowbuf, mbuf,
                  crow, prow, gsem, isem):
    wid = lax.axis_index("s") * NC + lax.axis_index("c")

    for ki in range(3):
        pltpu.sync_copy(cflat.at[pl.ds((ki + 1) * VOCAB, 1)],
                        crow.at[pl.ds(ki, 1)])
    pltpu.sync_copy(pos_hbm.at[pl.ds(0, 1)], prow)

    def bstep(i, _):
        b = wid * BPW + i
        pltpu.sync_copy(s3_hbm.at[b], sidx)
        pltpu.sync_copy(posx_hbm.at[b], posbuf)
        pltpu.sync_copy(n0_hbm.at[b], n0buf)
        pltpu.sync_copy(pz_hbm.at[b], pzbuf)
        pltpu.sync_copy(tot_hbm.at[b], totbuf)
        anypad = totbuf[pl.ds(0, L)][0] > 0.0

        for ki in range(3):
            pltpu.async_copy(pos_hbm.at[posbuf.at[pl.ds(0, 104)]],
                             mbuf.at[pl.ds(0, 104)], isem).wait()
            pltpu.async_copy(pos_hbm.at[posbuf.at[pl.ds(104, 96)]],
                             mbuf.at[pl.ds(104, 96)], isem).wait()

            row0 = ki * NCHUNK
            pltpu.async_copy(cflat.at[sidx.at[row0]], rowbuf.at[0], gsem)
            for c in range(NCHUNK):
                if c + 1 < NCHUNK:
                    pltpu.async_copy(cflat.at[sidx.at[row0 + c + 1]],
                                     rowbuf.at[(c + 1) % 2], gsem)
                pltpu.make_async_copy(cflat.at[sidx.at[row0 + c]],
                                      rowbuf.at[c % 2], gsem).wait()
                rb = rowbuf.at[c % 2]

                def mstep(ml, _, c=c, rb=rb):
                    mg = c * MPC + ml
                    r = 4 * ml
                    for j in range(DIM // L):
                        sl = pl.ds(L * j, L)
                        acc = rb[r, sl] + rb[r + 1, sl]
                        acc = acc + rb[r + 2, sl] + rb[r + 3, sl]
                        plsc.addupdate(mbuf.at[mg, sl], acc)
                    return 0

                lax.fori_loop(0, MPC, mstep, 0)

            @pl.when(anypad)
            def _fixup(ki=ki):
                def fstep(ml, _):
                    n0f = n0buf[pl.ds(ml, L)][0]
                    pzf = pzbuf[pl.ds(ml, L)][0]

                    @pl.when((n0f > 0.0) | (pzf > 0.0))
                    def _dofix():
                        n0v = jnp.broadcast_to(n0f, (L,))
                        pzv = jnp.broadcast_to(pzf, (L,))
                        for j in range(DIM // L):
                            sl = pl.ds(L * j, L)
                            cur = mbuf[ml, sl]
                            mbuf[ml, sl] = (cur - n0v * crow[ki, sl]
                                            - pzv * prow[0, sl])

                    return 0

                lax.fori_loop(0, MEM, fstep, 0)

            pltpu.sync_copy(mbuf, mout.at[ki, b])
        return 0

    lax.fori_loop(0, BPW, bstep, 0)


def _sc_pool(cflat, pos_table, s3, posx, n0, pz, tot):
    mesh = plsc.VectorSubcoreMesh(core_axis_name="c", subcore_axis_name="s",
                                  num_cores=NC, num_subcores=NS)
    f = pl.kernel(
        _sc_pool_body,
        out_type=jax.ShapeDtypeStruct((3, BATCH, MEM, DIM), jnp.float32),
        mesh=mesh,
        scratch_types=[
            pltpu.VMEM((3 * NCHUNK, CROWS), jnp.int32),
            pltpu.VMEM((MPAD,), jnp.int32),
            pltpu.VMEM((MPAD,), jnp.float32),
            pltpu.VMEM((MPAD,), jnp.float32),
            pltpu.VMEM((L,), jnp.float32),
            pltpu.VMEM((2, CROWS, DIM), jnp.float32),
            pltpu.VMEM((MEM, DIM), jnp.float32),
            pltpu.VMEM((3, DIM), jnp.float32),
            pltpu.VMEM((1, DIM), jnp.float32),
            pltpu.SemaphoreType.DMA,
            pltpu.SemaphoreType.DMA,
        ],
    )
    return f(cflat, pos_table, s3, posx, n0, pz, tot)


def _att_body(m1_ref, m2_ref, m3_ref, u_ref):
    m1 = m1_ref[0]
    m2 = m2_ref[0]
    m3 = m3_ref[0]
    u1 = jnp.mean(m1, axis=1)
    l1 = jnp.sum(m1 * u1[:, None, :], axis=2)
    p1 = jax.nn.softmax(l1, axis=1)
    u2 = u1 + jnp.sum(m2 * p1[:, :, None], axis=1)
    l2 = jnp.sum(m2 * u2[:, None, :], axis=2)
    p2 = jax.nn.softmax(l2, axis=1)
    u_ref[...] = u2 + jnp.sum(m3 * p2[:, :, None], axis=1)


def _attention(m):
    grid = (BATCH // _BB,)

    def spec(k):
        return pl.BlockSpec((1, _BB, MEM, DIM), lambda i, k=k: (k, i, 0, 0))

    return pl.pallas_call(
        _att_body,
        grid=grid,
        in_specs=[spec(0), spec(1), spec(2)],
        out_specs=pl.BlockSpec((_BB, DIM), lambda i: (i, 0)),
        out_shape=jax.ShapeDtypeStruct((BATCH, DIM), jnp.float32),
    )(m, m, m)


def kernel(C, pos_table, story):
    cflat = C.reshape(4 * VOCAB, DIM)
    s2 = jnp.transpose(story, (1, 0, 2)).reshape(BATCH, MEM * WIDTH)
    offs = (jnp.arange(1, 4, dtype=jnp.int32) * VOCAB)[None, :, None]
    s3 = (s2[:, None, :] + offs).reshape(BATCH, 3 * NCHUNK, CROWS)
    sm = jnp.pad(jnp.transpose(story, (1, 2, 0)),
                 ((0, 0), (0, 0), (0, MPAD - MEM)), constant_values=1)
    posx, n0, pz, tot = _prep(sm)
    m = _sc_pool(cflat, pos_table, s3, posx, n0, pz, tot)
    return _attention(m)


# speedup vs baseline: 11.2196x; 1.6043x over previous
import jax
import jax.numpy as jnp
from jax import lax
from jax.experimental import pallas as pl
from jax.experimental.pallas import tpu as pltpu
from jax.experimental.pallas import tpu_sc as plsc

VOCAB = 100000
DIM = 128
POS = 200
PAD = 0
MEM = 200
BATCH = 1024
WIDTH = 4

NC, NS, L = 2, 16, 16
NW = NC * NS
BPW = BATCH // NW
NCHUNK = 8
NT = 3 * NCHUNK
CROWS = MEM * WIDTH // NCHUNK
MPC = MEM // NCHUNK
MPAD = 224
PK_N0 = 0
PK_PZ = MPAD
PK_TOT = 2 * MPAD
PK_W = 2 * MPAD + 32

_PB = 128
_BB = 32


def _prep_body(sm_ref, pos_ref, n0_ref, pz_ref, tot_ref):
    sm = sm_ref[...]
    nonpad = sm[:, 0, :] != PAD
    npf = nonpad.astype(jnp.float32)
    ri = lax.broadcasted_iota(jnp.int32, (MPAD, MPAD), 0)
    ci = lax.broadcasted_iota(jnp.int32, (MPAD, MPAD), 1)
    tri = (ri <= ci).astype(jnp.float32)
    posf = jax.lax.dot_general(npf, tri, (((1,), (0,)), ((), ())),
                               preferred_element_type=jnp.float32)
    pos_ref[...] = jnp.where(nonpad, posf, 0.0).astype(jnp.int32)
    n0 = (sm == PAD).astype(jnp.float32).sum(axis=1)
    n0_ref[...] = n0
    pz_ref[...] = 1.0 - npf
    tot_ref[...] = jnp.broadcast_to(jnp.sum(n0, axis=1, keepdims=True),
                                    (sm.shape[0], 32))


def _prep(sm):
    grid = (BATCH // _PB,)
    out_sh = [
        jax.ShapeDtypeStruct((BATCH, MPAD), jnp.int32),
        jax.ShapeDtypeStruct((BATCH, MPAD), jnp.float32),
        jax.ShapeDtypeStruct((BATCH, MPAD), jnp.float32),
        jax.ShapeDtypeStruct((BATCH, 32), jnp.float32),
    ]
    ospec = pl.BlockSpec((_PB, MPAD), lambda i: (i, 0))
    return pl.pallas_call(
        _prep_body,
        grid=grid,
        in_specs=[pl.BlockSpec((_PB, WIDTH, MPAD), lambda i: (i, 0, 0))],
        out_specs=[ospec, ospec, ospec, pl.BlockSpec((_PB, 32), lambda i: (i, 0))],
        out_shape=out_sh,
    )(sm)


def _sc_pool_body(cflat, pos_hbm, s3_hbm, posx_hbm, fpack_hbm, mout,
                  sidx, posbuf, fpackbuf, rowbuf, mbuf, epbuf, crow, prow,
                  gsem, isem, osem):
    wid = lax.axis_index("s") * NC + lax.axis_index("c")

    for ki in range(3):
        pltpu.sync_copy(cflat.at[pl.ds((ki + 1) * VOCAB, 1)],
                        crow.at[pl.ds(ki, 1)])
    pltpu.sync_copy(pos_hbm.at[pl.ds(0, 1)], prow)

    def bstep(i, _):
        b = wid * BPW + i
        pltpu.sync_copy(s3_hbm.at[b], sidx)
        pltpu.sync_copy(posx_hbm.at[b], posbuf)
        pltpu.sync_copy(fpack_hbm.at[b], fpackbuf)
        anyv = fpackbuf[pl.ds(PK_TOT, L)][0] > 0.0

        pltpu.async_copy(pos_hbm.at[posbuf.at[pl.ds(0, 104)]],
                         epbuf.at[pl.ds(0, 104)], isem)
        pltpu.async_copy(pos_hbm.at[posbuf.at[pl.ds(104, 96)]],
                         epbuf.at[pl.ds(104, 96)], isem)

        pltpu.async_copy(cflat.at[sidx.at[0]], rowbuf.at[0], gsem)

        def tstep(t, _):
            c = t % NCHUNK
            ki = t // NCHUNK
            tb = t % 2
            mb = mbuf.at[ki % 2]

            @pl.when(t < NT - 1)
            def _issue():
                pltpu.async_copy(cflat.at[sidx.at[t + 1]],
                                 rowbuf.at[(t + 1) % 2], gsem)

            @pl.when(t == 2 * NCHUNK)
            def _wait_out0():
                pltpu.make_async_copy(mbuf.at[0], mout.at[0, b], osem).wait()

            pltpu.make_async_copy(cflat.at[sidx.at[t]], rowbuf.at[tb],
                                  gsem).wait()

            @plsc.parallel_loop(0, MPC, unroll=5)
            def _mstep(ml):
                mg = c * MPC + ml
                r = 4 * ml
                rb = rowbuf.at[tb]
                for j in range(DIM // L):
                    sl = pl.ds(L * j, L)
                    acc = (rb[r, sl] + rb[r + 1, sl]) + (rb[r + 2, sl]
                                                         + rb[r + 3, sl])
                    mb[mg, sl] = acc

            @pl.when(c == NCHUNK - 1)
            def _finish():
                @pl.when(anyv)
                def _fx():
                    def fstep(ml, _):
                        n0f = fpackbuf[pl.ds(PK_N0 + ml, L)][0]

                        @pl.when(n0f > 0.0)
                        def _dofix():
                            n0v = jnp.broadcast_to(n0f, (L,))
                            for j in range(DIM // L):
                                sl = pl.ds(L * j, L)
                                mb[ml, sl] = mb[ml, sl] - n0v * crow[ki, sl]

                        return 0

                    lax.fori_loop(0, MEM, fstep, 0)

                pltpu.async_copy(mb, mout.at[ki, b], osem)

            return 0

        lax.fori_loop(0, NT, tstep, 0)

        pltpu.make_async_copy(pos_hbm.at[posbuf.at[pl.ds(0, 104)]],
                              epbuf.at[pl.ds(0, 104)], isem).wait()
        pltpu.make_async_copy(pos_hbm.at[posbuf.at[pl.ds(104, 96)]],
                              epbuf.at[pl.ds(104, 96)], isem).wait()

        @pl.when(anyv)
        def _epfix():
            def fstep(ml, _):
                pzf = fpackbuf[pl.ds(PK_PZ + ml, L)][0]

                @pl.when(pzf > 0.0)
                def _dofix():
                    pzv = jnp.broadcast_to(pzf, (L,))
                    for j in range(DIM // L):
                        sl = pl.ds(L * j, L)
                        epbuf[ml, sl] = epbuf[ml, sl] - pzv * prow[0, sl]

                return 0

            lax.fori_loop(0, MEM, fstep, 0)

        pltpu.async_copy(epbuf, mout.at[3, b], osem)

        for _ in range(3):
            pltpu.make_async_copy(mbuf.at[0], mout.at[0, b], osem).wait()
        return 0

    lax.fori_loop(0, BPW, bstep, 0)


def _sc_pool(cflat, pos_table, s3, posx, fpack):
    mesh = plsc.VectorSubcoreMesh(core_axis_name="c", subcore_axis_name="s",
                                  num_cores=NC, num_subcores=NS)
    f = pl.kernel(
        _sc_pool_body,
        out_type=jax.ShapeDtypeStruct((4, BATCH, MEM, DIM), jnp.float32),
        mesh=mesh,
        scratch_types=[
            pltpu.VMEM((NT, CROWS), jnp.int32),
            pltpu.VMEM((MPAD,), jnp.int32),
            pltpu.VMEM((PK_W,), jnp.float32),
            pltpu.VMEM((2, CROWS, DIM), jnp.float32),
            pltpu.VMEM((2, MEM, DIM), jnp.float32),
            pltpu.VMEM((MEM, DIM), jnp.float32),
            pltpu.VMEM((3, DIM), jnp.float32),
            pltpu.VMEM((1, DIM), jnp.float32),
            pltpu.SemaphoreType.DMA,
            pltpu.SemaphoreType.DMA,
            pltpu.SemaphoreType.DMA,
        ],
    )
    return f(cflat, pos_table, s3, posx, fpack)


def _att_body(m1_ref, m2_ref, m3_ref, ep_ref, u_ref):
    ep = ep_ref[0]
    m1 = m1_ref[0] + ep
    m2 = m2_ref[0] + ep
    m3 = m3_ref[0] + ep
    u1 = jnp.mean(m1, axis=1)
    l1 = jnp.sum(m1 * u1[:, None, :], axis=2)
    p1 = jax.nn.softmax(l1, axis=1)
    u2 = u1 + jnp.sum(m2 * p1[:, :, None], axis=1)
    l2 = jnp.sum(m2 * u2[:, None, :], axis=2)
    p2 = jax.nn.softmax(l2, axis=1)
    u_ref[...] = u2 + jnp.sum(m3 * p2[:, :, None], axis=1)


def _attention(m):
    grid = (BATCH // _BB,)

    def spec(k):
        return pl.BlockSpec((1, _BB, MEM, DIM), lambda i, k=k: (k, i, 0, 0))

    return pl.pallas_call(
        _att_body,
        grid=grid,
        in_specs=[spec(0), spec(1), spec(2), spec(3)],
        out_specs=pl.BlockSpec((_BB, DIM), lambda i: (i, 0)),
        out_shape=jax.ShapeDtypeStruct((BATCH, DIM), jnp.float32),
    )(m, m, m, m)


def kernel(C, pos_table, story):
    cflat = C.reshape(4 * VOCAB, DIM)
    s2 = jnp.transpose(story, (1, 0, 2)).reshape(BATCH, MEM * WIDTH)
    offs = (jnp.arange(1, 4, dtype=jnp.int32) * VOCAB)[None, :, None]
    s3 = (s2[:, None, :] + offs).reshape(BATCH, NT, CROWS)
    sm = jnp.pad(jnp.transpose(story, (1, 2, 0)),
                 ((0, 0), (0, 0), (0, MPAD - MEM)), constant_values=1)
    posx, n0, pz, tot = _prep(sm)
    fpack = jnp.concatenate([n0, pz, tot], axis=1)
    m = _sc_pool(cflat, pos_table, s3, posx, fpack)
    return _attention(m)


# speedup vs baseline: 11.6793x; 1.0410x over previous
import jax
import jax.numpy as jnp
from jax import lax
from jax.experimental import pallas as pl
from jax.experimental.pallas import tpu as pltpu
from jax.experimental.pallas import tpu_sc as plsc

VOCAB = 100000
DIM = 128
POS = 200
PAD = 0
MEM = 200
BATCH = 1024
WIDTH = 4

NC, NS, L = 2, 16, 16
NW = NC * NS
BPW = BATCH // NW
NCHUNK = 8
NT = 3 * NCHUNK
CROWS = MEM * WIDTH // NCHUNK
MPC = MEM // NCHUNK
MPAD = 224
PK_N0 = 0
PK_PZ = MPAD
PK_TOT = 2 * MPAD
PK_W = 2 * MPAD + 32

_PB = 128
_BB = 32


def _prep_body(sm_ref, pos_ref, n0_ref, pz_ref, tot_ref):
    sm = sm_ref[...]
    nonpad = sm[:, 0, :] != PAD
    npf = nonpad.astype(jnp.float32)
    ri = lax.broadcasted_iota(jnp.int32, (MPAD, MPAD), 0)
    ci = lax.broadcasted_iota(jnp.int32, (MPAD, MPAD), 1)
    tri = (ri <= ci).astype(jnp.float32)
    posf = jax.lax.dot_general(npf, tri, (((1,), (0,)), ((), ())),
                               preferred_element_type=jnp.float32)
    pos_ref[...] = jnp.where(nonpad, posf, 0.0).astype(jnp.int32)
    n0 = (sm == PAD).astype(jnp.float32).sum(axis=1)
    n0_ref[...] = n0
    pz_ref[...] = 1.0 - npf
    tot_ref[...] = jnp.broadcast_to(jnp.sum(n0, axis=1, keepdims=True),
                                    (sm.shape[0], 32))


def _prep(sm):
    grid = (BATCH // _PB,)
    out_sh = [
        jax.ShapeDtypeStruct((BATCH, MPAD), jnp.int32),
        jax.ShapeDtypeStruct((BATCH, MPAD), jnp.float32),
        jax.ShapeDtypeStruct((BATCH, MPAD), jnp.float32),
        jax.ShapeDtypeStruct((BATCH, 32), jnp.float32),
    ]
    ospec = pl.BlockSpec((_PB, MPAD), lambda i: (i, 0))
    return pl.pallas_call(
        _prep_body,
        grid=grid,
        in_specs=[pl.BlockSpec((_PB, WIDTH, MPAD), lambda i: (i, 0, 0))],
        out_specs=[ospec, ospec, ospec, pl.BlockSpec((_PB, 32), lambda i: (i, 0))],
        out_shape=out_sh,
    )(sm)


def _sc_pool_body(cflat, pos_hbm, s3_hbm, posx_hbm, fpack_hbm, mout,
                  sidx, posbuf, fpackbuf, rowbuf, mbuf, epbuf, crow, prow,
                  gsem, isem, osem, eosem):
    wid = lax.axis_index("s") * NC + lax.axis_index("c")

    for ki in range(3):
        pltpu.sync_copy(cflat.at[pl.ds((ki + 1) * VOCAB, 1)],
                        crow.at[pl.ds(ki, 1)])
    pltpu.sync_copy(pos_hbm.at[pl.ds(0, 1)], prow)

    b0 = wid * BPW
    pltpu.sync_copy(s3_hbm.at[b0], sidx.at[0])
    pltpu.async_copy(cflat.at[sidx.at[0, 0]], rowbuf.at[0], gsem)

    def bstep(i, _):
        b = wid * BPW + i
        sb = i % 2
        @pl.when(i > 0)
        def _drain():
            pltpu.make_async_copy(mbuf.at[0], mout.at[0, b], osem).wait()
            pltpu.make_async_copy(mbuf.at[0], mout.at[0, b], osem).wait()
            pltpu.make_async_copy(epbuf, mout.at[3, b], eosem).wait()

        pltpu.sync_copy(posx_hbm.at[b], posbuf)
        pltpu.sync_copy(fpack_hbm.at[b], fpackbuf)
        anyv = fpackbuf[pl.ds(PK_TOT, L)][0] > 0.0

        pltpu.async_copy(pos_hbm.at[posbuf.at[pl.ds(0, 104)]],
                         epbuf.at[pl.ds(0, 104)], isem)
        pltpu.async_copy(pos_hbm.at[posbuf.at[pl.ds(104, 96)]],
                         epbuf.at[pl.ds(104, 96)], isem)

        def tstep(t, _):
            c = t % NCHUNK
            ki = t // NCHUNK
            tb = t % 2
            mb = mbuf.at[ki % 2]

            @pl.when(t < NT - 1)
            def _issue():
                pltpu.async_copy(cflat.at[sidx.at[sb, t + 1]],
                                 rowbuf.at[(t + 1) % 2], gsem)

            @pl.when(t == 2 * NCHUNK)
            def _wait_out0():
                pltpu.make_async_copy(mbuf.at[0], mout.at[0, b], osem).wait()

            pltpu.make_async_copy(cflat.at[sidx.at[sb, t]], rowbuf.at[tb],
                                  gsem).wait()

            @plsc.parallel_loop(0, MPC, unroll=5)
            def _mstep(ml):
                mg = c * MPC + ml
                r = 4 * ml
                rb = rowbuf.at[tb]
                for j in range(DIM // L):
                    sl = pl.ds(L * j, L)
                    acc = (rb[r, sl] + rb[r + 1, sl]) + (rb[r + 2, sl]
                                                         + rb[r + 3, sl])
                    mb[mg, sl] = acc

            @pl.when(c == NCHUNK - 1)
            def _finish():
                @pl.when(anyv)
                def _fx():
                    def fstep(ml, _):
                        n0f = fpackbuf[pl.ds(PK_N0 + ml, L)][0]

                        @pl.when(n0f > 0.0)
                        def _dofix():
                            n0v = jnp.broadcast_to(n0f, (L,))
                            for j in range(DIM // L):
                                sl = pl.ds(L * j, L)
                                mb[ml, sl] = mb[ml, sl] - n0v * crow[ki, sl]

                        return 0

                    lax.fori_loop(0, MEM, fstep, 0)

                pltpu.async_copy(mb, mout.at[ki, b], osem)

                @pl.when((ki == 2) & (i < BPW - 1))
                def _pre():
                    pltpu.sync_copy(s3_hbm.at[b + 1], sidx.at[1 - sb])
                    pltpu.async_copy(cflat.at[sidx.at[1 - sb, 0]],
                                     rowbuf.at[0], gsem)

            return 0

        lax.fori_loop(0, NT, tstep, 0)

        pltpu.make_async_copy(pos_hbm.at[posbuf.at[pl.ds(0, 104)]],
                              epbuf.at[pl.ds(0, 104)], isem).wait()
        pltpu.make_async_copy(pos_hbm.at[posbuf.at[pl.ds(104, 96)]],
                              epbuf.at[pl.ds(104, 96)], isem).wait()

        @pl.when(anyv)
        def _epfix():
            def fstep(ml, _):
                pzf = fpackbuf[pl.ds(PK_PZ + ml, L)][0]

                @pl.when(pzf > 0.0)
                def _dofix():
                    pzv = jnp.broadcast_to(pzf, (L,))
                    for j in range(DIM // L):
                        sl = pl.ds(L * j, L)
                        epbuf[ml, sl] = epbuf[ml, sl] - pzv * prow[0, sl]

                return 0

            lax.fori_loop(0, MEM, fstep, 0)

        pltpu.async_copy(epbuf, mout.at[3, b], eosem)
        return 0

    lax.fori_loop(0, BPW, bstep, 0)
    pltpu.make_async_copy(mbuf.at[0], mout.at[0, 0], osem).wait()
    pltpu.make_async_copy(mbuf.at[0], mout.at[0, 0], osem).wait()
    pltpu.make_async_copy(epbuf, mout.at[3, 0], eosem).wait()


def _sc_pool(cflat, pos_table, s3, posx, fpack):
    mesh = plsc.VectorSubcoreMesh(core_axis_name="c", subcore_axis_name="s",
                                  num_cores=NC, num_subcores=NS)
    f = pl.kernel(
        _sc_pool_body,
        out_type=jax.ShapeDtypeStruct((4, BATCH, MEM, DIM), jnp.float32),
        mesh=mesh,
        scratch_types=[
            pltpu.VMEM((2, NT, CROWS), jnp.int32),
            pltpu.VMEM((MPAD,), jnp.int32),
            pltpu.VMEM((PK_W,), jnp.float32),
            pltpu.VMEM((2, CROWS, DIM), jnp.float32),
            pltpu.VMEM((2, MEM, DIM), jnp.float32),
            pltpu.VMEM((MEM, DIM), jnp.float32),
            pltpu.VMEM((3, DIM), jnp.float32),
            pltpu.VMEM((1, DIM), jnp.float32),
            pltpu.SemaphoreType.DMA,
            pltpu.SemaphoreType.DMA,
            pltpu.SemaphoreType.DMA,
            pltpu.SemaphoreType.DMA,
        ],
    )
    return f(cflat, pos_table, s3, posx, fpack)


def _att_body(m1_ref, m2_ref, m3_ref, ep_ref, u_ref):
    ep = ep_ref[0]
    m1 = m1_ref[0] + ep
    m2 = m2_ref[0] + ep
    m3 = m3_ref[0] + ep
    u1 = jnp.mean(m1, axis=1)
    l1 = jnp.sum(m1 * u1[:, None, :], axis=2)
    p1 = jax.nn.softmax(l1, axis=1)
    u2 = u1 + jnp.sum(m2 * p1[:, :, None], axis=1)
    l2 = jnp.sum(m2 * u2[:, None, :], axis=2)
    p2 = jax.nn.softmax(l2, axis=1)
    u_ref[...] = u2 + jnp.sum(m3 * p2[:, :, None], axis=1)


def _attention(m):
    grid = (BATCH // _BB,)

    def spec(k):
        return pl.BlockSpec((1, _BB, MEM, DIM), lambda i, k=k: (k, i, 0, 0))

    return pl.pallas_call(
        _att_body,
        grid=grid,
        in_specs=[spec(0), spec(1), spec(2), spec(3)],
        out_specs=pl.BlockSpec((_BB, DIM), lambda i: (i, 0)),
        out_shape=jax.ShapeDtypeStruct((BATCH, DIM), jnp.float32),
    )(m, m, m, m)


def kernel(C, pos_table, story):
    cflat = C.reshape(4 * VOCAB, DIM)
    s2 = jnp.transpose(story, (1, 0, 2)).reshape(BATCH, MEM * WIDTH)
    offs = (jnp.arange(1, 4, dtype=jnp.int32) * VOCAB)[None, :, None]
    s3 = (s2[:, None, :] + offs).reshape(BATCH, NT, CROWS)
    sm = jnp.pad(jnp.transpose(story, (1, 2, 0)),
                 ((0, 0), (0, 0), (0, MPAD - MEM)), constant_values=1)
    posx, n0, pz, tot = _prep(sm)
    fpack = jnp.concatenate([n0, pz, tot], axis=1)
    m = _sc_pool(cflat, pos_table, s3, posx, fpack)
    return _attention(m)


# speedup vs baseline: 13.0112x; 1.1140x over previous
import jax
import jax.numpy as jnp
from jax import lax
from jax.experimental import pallas as pl
from jax.experimental.pallas import tpu as pltpu
from jax.experimental.pallas import tpu_sc as plsc

VOCAB = 100000
DIM = 128
POS = 200
PAD = 0
MEM = 200
BATCH = 1024
WIDTH = 4

NC, NS, L = 2, 16, 16
NW = NC * NS
BPW = BATCH // NW
NCHUNK = 8
NT = 3 * NCHUNK
CROWS = MEM * WIDTH // NCHUNK
MPC = MEM // NCHUNK
MPAD = 224
PK_N0 = 0
PK_PZ = MPAD
PK_TOT = 2 * MPAD
PK_W = 2 * MPAD + 32

_PB = 128
_BB = 32


def _prep_body(sm_ref, pos_ref, n0_ref, pz_ref, tot_ref):
    sm = sm_ref[...]
    nonpad = sm[:, 0, :] != PAD
    npf = nonpad.astype(jnp.float32)
    ri = lax.broadcasted_iota(jnp.int32, (MPAD, MPAD), 0)
    ci = lax.broadcasted_iota(jnp.int32, (MPAD, MPAD), 1)
    tri = (ri <= ci).astype(jnp.float32)
    posf = jax.lax.dot_general(npf, tri, (((1,), (0,)), ((), ())),
                               preferred_element_type=jnp.float32)
    pos_ref[...] = jnp.where(nonpad, posf, 0.0).astype(jnp.int32)
    n0 = (sm == PAD).astype(jnp.float32).sum(axis=1)
    n0_ref[...] = n0
    pz_ref[...] = 1.0 - npf
    tot_ref[...] = jnp.broadcast_to(jnp.sum(n0, axis=1, keepdims=True),
                                    (sm.shape[0], 32))


def _prep(sm):
    grid = (BATCH // _PB,)
    out_sh = [
        jax.ShapeDtypeStruct((BATCH, MPAD), jnp.int32),
        jax.ShapeDtypeStruct((BATCH, MPAD), jnp.float32),
        jax.ShapeDtypeStruct((BATCH, MPAD), jnp.float32),
        jax.ShapeDtypeStruct((BATCH, 32), jnp.float32),
    ]
    ospec = pl.BlockSpec((_PB, MPAD), lambda i: (i, 0))
    return pl.pallas_call(
        _prep_body,
        grid=grid,
        in_specs=[pl.BlockSpec((_PB, WIDTH, MPAD), lambda i: (i, 0, 0))],
        out_specs=[ospec, ospec, ospec, pl.BlockSpec((_PB, 32), lambda i: (i, 0))],
        out_shape=out_sh,
    )(sm)


def _sc_pool_body(cflat, s3_hbm, fpack_hbm, mout,
                  sidx, fpackbuf, rowbuf, mbuf, crow,
                  gsem, osem):
    wid = lax.axis_index("s") * NC + lax.axis_index("c")

    for ki in range(3):
        pltpu.sync_copy(cflat.at[pl.ds((ki + 1) * VOCAB, 1)],
                        crow.at[pl.ds(ki, 1)])

    b0 = wid * BPW
    pltpu.sync_copy(s3_hbm.at[b0], sidx.at[0])
    pltpu.async_copy(cflat.at[sidx.at[0, 0]], rowbuf.at[0], gsem)

    def bstep(i, _):
        b = wid * BPW + i
        sb = i % 2
        @pl.when(i > 0)
        def _drain():
            pltpu.make_async_copy(mbuf.at[0], mout.at[0, b], osem).wait()
            pltpu.make_async_copy(mbuf.at[0], mout.at[0, b], osem).wait()

        pltpu.sync_copy(fpack_hbm.at[b], fpackbuf)
        anyv = fpackbuf[pl.ds(PK_TOT, L)][0] > 0.0

        def tstep(t, _):
            c = t % NCHUNK
            ki = t // NCHUNK
            tb = t % 2
            mb = mbuf.at[ki % 2]

            @pl.when(t < NT - 1)
            def _issue():
                pltpu.async_copy(cflat.at[sidx.at[sb, t + 1]],
                                 rowbuf.at[(t + 1) % 2], gsem)

            @pl.when(t == 2 * NCHUNK)
            def _wait_out0():
                pltpu.make_async_copy(mbuf.at[0], mout.at[0, b], osem).wait()

            pltpu.make_async_copy(cflat.at[sidx.at[sb, t]], rowbuf.at[tb],
                                  gsem).wait()

            @plsc.parallel_loop(0, MPC, unroll=25)
            def _mstep(ml):
                mg = c * MPC + ml
                r = 4 * ml
                rb = rowbuf.at[tb]
                for j in range(DIM // L):
                    sl = pl.ds(L * j, L)
                    acc = (rb[r, sl] + rb[r + 1, sl]) + (rb[r + 2, sl]
                                                         + rb[r + 3, sl])
                    mb[mg, sl] = acc

            @pl.when(c == NCHUNK - 1)
            def _finish():
                @pl.when(anyv)
                def _fx():
                    def fstep(ml, _):
                        n0f = fpackbuf[pl.ds(PK_N0 + ml, L)][0]

                        @pl.when(n0f > 0.0)
                        def _dofix():
                            n0v = jnp.broadcast_to(n0f, (L,))
                            for j in range(DIM // L):
                                sl = pl.ds(L * j, L)
                                mb[ml, sl] = mb[ml, sl] - n0v * crow[ki, sl]

                        return 0

                    lax.fori_loop(0, MEM, fstep, 0)

                pltpu.async_copy(mb, mout.at[ki, b], osem)

                @pl.when((ki == 2) & (i < BPW - 1))
                def _pre():
                    pltpu.sync_copy(s3_hbm.at[b + 1], sidx.at[1 - sb])
                    pltpu.async_copy(cflat.at[sidx.at[1 - sb, 0]],
                                     rowbuf.at[0], gsem)

            return 0

        lax.fori_loop(0, NT, tstep, 0)
        return 0

    lax.fori_loop(0, BPW, bstep, 0)
    pltpu.make_async_copy(mbuf.at[0], mout.at[0, 0], osem).wait()
    pltpu.make_async_copy(mbuf.at[0], mout.at[0, 0], osem).wait()


def _sc_pool(cflat, s3, fpack):
    mesh = plsc.VectorSubcoreMesh(core_axis_name="c", subcore_axis_name="s",
                                  num_cores=NC, num_subcores=NS)
    f = pl.kernel(
        _sc_pool_body,
        out_type=jax.ShapeDtypeStruct((3, BATCH, MEM, DIM), jnp.float32),
        mesh=mesh,
        scratch_types=[
            pltpu.VMEM((2, NT, CROWS), jnp.int32),
            pltpu.VMEM((PK_W,), jnp.float32),
            pltpu.VMEM((2, CROWS, DIM), jnp.float32),
            pltpu.VMEM((2, MEM, DIM), jnp.float32),
            pltpu.VMEM((3, DIM), jnp.float32),
            pltpu.SemaphoreType.DMA,
            pltpu.SemaphoreType.DMA,
        ],
    )
    return f(cflat, s3, fpack)


def _att_body(m1_ref, m2_ref, m3_ref, posx_ref, pt_ref, u_ref):
    pos = posx_ref[...][:, :MEM]
    pos3 = pos[:, :, None]
    iota3 = lax.broadcasted_iota(jnp.int32, (_BB, MEM, MPAD), 2)
    oh = ((pos3 == iota3) & (pos3 > 0)).astype(jnp.float32)
    ep = jax.lax.dot_general(oh, pt_ref[...], (((2,), (0,)), ((), ())),
                             preferred_element_type=jnp.float32)
    m1 = m1_ref[0] + ep
    m2 = m2_ref[0] + ep
    m3 = m3_ref[0] + ep
    u1 = jnp.mean(m1, axis=1)
    l1 = jnp.sum(m1 * u1[:, None, :], axis=2)
    p1 = jax.nn.softmax(l1, axis=1)
    u2 = u1 + jnp.sum(m2 * p1[:, :, None], axis=1)
    l2 = jnp.sum(m2 * u2[:, None, :], axis=2)
    p2 = jax.nn.softmax(l2, axis=1)
    u_ref[...] = u2 + jnp.sum(m3 * p2[:, :, None], axis=1)


def _attention(m, posx, ptp):
    grid = (BATCH // _BB,)

    def spec(k):
        return pl.BlockSpec((1, _BB, MEM, DIM), lambda i, k=k: (k, i, 0, 0))

    return pl.pallas_call(
        _att_body,
        grid=grid,
        in_specs=[
            spec(0), spec(1), spec(2),
            pl.BlockSpec((_BB, MPAD), lambda i: (i, 0)),
            pl.BlockSpec((MPAD, DIM), lambda i: (0, 0)),
        ],
        out_specs=pl.BlockSpec((_BB, DIM), lambda i: (i, 0)),
        out_shape=jax.ShapeDtypeStruct((BATCH, DIM), jnp.float32),
    )(m, m, m, posx, ptp)


def kernel(C, pos_table, story):
    cflat = C.reshape(4 * VOCAB, DIM)
    s2 = jnp.transpose(story, (1, 0, 2)).reshape(BATCH, MEM * WIDTH)
    offs = (jnp.arange(1, 4, dtype=jnp.int32) * VOCAB)[None, :, None]
    s3 = (s2[:, None, :] + offs).reshape(BATCH, NT, CROWS)
    sm = jnp.pad(jnp.transpose(story, (1, 2, 0)),
                 ((0, 0), (0, 0), (0, MPAD - MEM)), constant_values=1)
    posx, n0, pz, tot = _prep(sm)
    fpack = jnp.concatenate([n0, pz, tot], axis=1)
    ptp = jnp.pad(pos_table, ((0, MPAD - (POS + 1)), (0, 0)))
    m = _sc_pool(cflat, s3, fpack)
    return _attention(m, posx, ptp)
